# baseline (speedup 1.0000x reference)
import functools

import jax
import jax.numpy as jnp
from jax import lax
from jax.experimental import pallas as pl
from jax.experimental.pallas import tpu as pltpu
from jax.experimental.pallas import tpu_sc as plsc

_NUM_CORES = 2
_NUM_SUBCORES = 16
_NW = _NUM_CORES * _NUM_SUBCORES
_CHUNK = 128


def kernel(srcs, dsts, table):
    B = srcs.shape[0]
    D = table.shape[1]
    rows_per_w = B // _NW
    chunks_per_w = rows_per_w // _CHUNK

    srcs2 = srcs.reshape(B // _CHUNK, _CHUNK)
    dsts2 = dsts.reshape(B // _CHUNK, _CHUNK)

    mesh = plsc.VectorSubcoreMesh(
        core_axis_name="c", subcore_axis_name="s",
        num_cores=_NUM_CORES, num_subcores=_NUM_SUBCORES)

    @functools.partial(
        pl.kernel,
        out_type=(jax.ShapeDtypeStruct((B, D), jnp.float32),
                  jax.ShapeDtypeStruct((B, D), jnp.float32)),
        mesh=mesh,
        scratch_types=[
            pltpu.VMEM((chunks_per_w, _CHUNK), jnp.int32),
            pltpu.VMEM((chunks_per_w, _CHUNK), jnp.int32),
            pltpu.VMEM((rows_per_w, D), jnp.float32),
            pltpu.VMEM((rows_per_w, D), jnp.float32),
            pltpu.SemaphoreType.DMA,
        ],
        compiler_params=pltpu.CompilerParams(use_tc_tiling_on_sc=False),
    )
    def deepwalk_lookup(srcs_hbm, dsts_hbm, table_hbm, out_s, out_d,
                        idx_s, idx_d, rows_s, rows_d, sem):
        wid = lax.axis_index("s") * _NUM_CORES + lax.axis_index("c")
        crow = wid * chunks_per_w
        pltpu.sync_copy(srcs_hbm.at[pl.ds(crow, chunks_per_w)], idx_s)
        pltpu.sync_copy(dsts_hbm.at[pl.ds(crow, chunks_per_w)], idx_d)
        copies = []
        for j in range(chunks_per_w):
            copies.append(pltpu.async_copy(
                table_hbm.at[idx_s.at[j]],
                rows_s.at[pl.ds(j * _CHUNK, _CHUNK)], sem))
            copies.append(pltpu.async_copy(
                table_hbm.at[idx_d.at[j]],
                rows_d.at[pl.ds(j * _CHUNK, _CHUNK)], sem))
        for c in copies:
            c.wait()
        base = wid * rows_per_w
        pltpu.sync_copy(rows_s, out_s.at[pl.ds(base, rows_per_w)])
        pltpu.sync_copy(rows_d, out_d.at[pl.ds(base, rows_per_w)])

    return deepwalk_lookup(srcs2, dsts2, table)


# speedup vs baseline: 1.6678x; 1.6678x over previous
import functools

import jax
import jax.numpy as jnp
from jax import lax
from jax.experimental import pallas as pl
from jax.experimental.pallas import tpu as pltpu
from jax.experimental.pallas import tpu_sc as plsc

_NUM_CORES = 2
_NUM_SUBCORES = 16
_NW = _NUM_CORES * _NUM_SUBCORES
_CHUNK = 128
_LANES = 16


def kernel(srcs, dsts, table):
    B = srcs.shape[0]
    D = table.shape[1]
    rows_per_w = B // _NW
    n_chunks = rows_per_w // _CHUNK

    srcs2 = srcs.reshape(B // _CHUNK, _CHUNK)
    dsts2 = dsts.reshape(B // _CHUNK, _CHUNK)

    mesh = plsc.VectorSubcoreMesh(
        core_axis_name="c", subcore_axis_name="s",
        num_cores=_NUM_CORES, num_subcores=_NUM_SUBCORES)

    @functools.partial(
        pl.kernel,
        out_type=(jax.ShapeDtypeStruct((B, D), jnp.float32),
                  jax.ShapeDtypeStruct((B, D), jnp.float32)),
        mesh=mesh,
        scratch_types=[
            pltpu.VMEM((n_chunks, _CHUNK), jnp.int32),
            pltpu.VMEM((n_chunks, _CHUNK), jnp.int32),
            pltpu.VMEM((_CHUNK, D), jnp.float32),
            pltpu.VMEM((_CHUNK, D), jnp.float32),
            pltpu.SemaphoreType.DMA,
        ],
    )
    def deepwalk_lookup(srcs_hbm, dsts_hbm, table_hbm, out_s, out_d,
                        idx_s, idx_d, rows_s, rows_d, sem):
        wid = lax.axis_index("s") * _NUM_CORES + lax.axis_index("c")
        crow = wid * n_chunks
        pltpu.sync_copy(srcs_hbm.at[pl.ds(crow, n_chunks)], idx_s)
        pltpu.sync_copy(dsts_hbm.at[pl.ds(crow, n_chunks)], idx_d)
        base = wid * rows_per_w

        def run_list(idx_ref, rows_ref, out_ref):
            def chunk_body(c, carry):
                copies = []
                for r in range(_CHUNK // _LANES):
                    v = idx_ref[c, pl.ds(r * _LANES, _LANES)]
                    for l in range(_LANES):
                        i = v[l]
                        copies.append(pltpu.async_copy(
                            table_hbm.at[i], rows_ref.at[r * _LANES + l],
                            sem))
                for cp in copies:
                    cp.wait()
                pltpu.sync_copy(
                    rows_ref, out_ref.at[pl.ds(base + c * _CHUNK, _CHUNK)])
                return carry

            lax.fori_loop(0, n_chunks, chunk_body, 0)

        run_list(idx_s, rows_s, out_s)
        run_list(idx_d, rows_d, out_d)

    return deepwalk_lookup(srcs2, dsts2, table)
